# baseline (speedup 1.0000x reference)
import jax
import jax.numpy as jnp
from jax import lax
from jax.experimental import pallas as pl
from jax.experimental.pallas import tpu as pltpu

_C = 384
_NS = 1024
_WS = 4
_WNH = 8
_NW = 64
_K = 16
_G = 24
_GD = 4 * _G
_NT = _K * _WS * _WS
_NB = 8

_RHS_T = (((1,), (1,)), ((), ()))


def _body(z_ref, x_ref, wd_ref, bd_ref, wu_ref, bu_ref, out_ref, xe_ref,
          sem):
    ti = lax.broadcasted_iota(jnp.int32, (_NW, _NS), 1)
    wi = lax.broadcasted_iota(jnp.int32, (_NW, _NS), 0)
    q_sel = ((ti // 128 == wi // _WNH)
             & ((ti % 32) // _WS == wi % _WNH)).astype(jnp.float32)
    eye = (lax.broadcasted_iota(jnp.int32, (_NW, _NW), 0)
           == lax.broadcasted_iota(jnp.int32, (_NW, _NW), 1)).astype(
        jnp.float32)
    wi64 = lax.broadcasted_iota(jnp.int32, (_NW, _NW), 0)
    ji64 = lax.broadcasted_iota(jnp.int32, (_NW, _NW), 1)
    fidx = lax.broadcasted_iota(jnp.int32, (_NW, 1), 0)

    copies = []
    for i in range(_NB):
        z = z_ref[i]
        zmax = jnp.max(z, axis=0, keepdims=True)
        sim = lax.dot_general(zmax, x_ref[i], _RHS_T,
                              preferred_element_type=jnp.float32)
        win_row = lax.dot_general(sim, q_sel, _RHS_T,
                                  preferred_element_type=jnp.float32,
                                  precision=lax.Precision.HIGHEST)
        win_col = lax.dot_general(eye, win_row, _RHS_T,
                                  preferred_element_type=jnp.float32,
                                  precision=lax.Precision.HIGHEST)
        vj = jnp.broadcast_to(win_row, (_NW, _NW))
        vw = jnp.broadcast_to(win_col, (_NW, _NW))
        beats = (vj > vw) | ((vj == vw) & (ji64 < wi64))
        rank = jnp.sum(beats.astype(jnp.int32), axis=1, keepdims=True)
        for k in range(_K):
            w = jnp.sum(jnp.where(rank == k, fidx, 0))
            wr = w // _WNH
            base = wr * 128 + (w - wr * _WNH) * _WS
            row0 = i * _NT + _K * k
            for r in range(_WS):
                cp = pltpu.make_async_copy(
                    x_ref.at[i, pl.ds(base + 32 * r, _WS), :],
                    xe_ref.at[pl.ds(row0 + _WS * r, _WS), :],
                    sem)
                cp.start()
                copies.append(cp)
    for cp in copies:
        cp.wait()

    rows = _NB * _NT
    xe = xe_ref[...]
    wd = wd_ref[...]
    bd = bd_ref[...]
    wu = wu_ref[...]
    bu = bu_ref[...]

    t = lax.dot_general(xe, wd, _RHS_T,
                        preferred_element_type=jnp.float32) + bd

    qi = lax.broadcasted_iota(jnp.int32, (rows, _GD), 0)
    li = lax.broadcasted_iota(jnp.int32, (rows, _GD), 1)
    c_tok = qi % _WS
    r_tok = (qi // _WS) % _WS
    z1 = jnp.zeros((1, _GD), jnp.float32)
    z4 = jnp.zeros((_WS, _GD), jnp.float32)
    tm1 = jnp.concatenate([t[1:], z1], axis=0)
    tp1 = jnp.concatenate([z1, t[:-1]], axis=0)
    tm4 = jnp.concatenate([t[_WS:], z4], axis=0)
    tp4 = jnp.concatenate([z4, t[:-_WS]], axis=0)
    g0 = jnp.where(c_tok < _WS - 1, tm1, 0.0)
    g1 = jnp.where(c_tok > 0, tp1, 0.0)
    g2 = jnp.where(r_tok < _WS - 1, tm4, 0.0)
    g3 = jnp.where(r_tok > 0, tp4, 0.0)
    s = jnp.where(li < _G, g0,
                  jnp.where(li < 2 * _G, g1,
                            jnp.where(li < 3 * _G, g2, g3)))
    su = lax.dot_general(s, wu, _RHS_T,
                         preferred_element_type=jnp.float32)
    out = xe + su + bu
    for i in range(_NB):
        out_ref[i] = out[i * _NT:(i + 1) * _NT]


def kernel(z, x, w_down, b_down, w_up, b_up):
    B = z.shape[0]
    bd = b_down.reshape(1, _GD)
    bu = b_up.reshape(1, _C)
    return pl.pallas_call(
        _body,
        grid=(B // _NB,),
        in_specs=[
            pl.BlockSpec((_NB, z.shape[1], _C), lambda b: (b, 0, 0)),
            pl.BlockSpec((_NB, _NS, _C), lambda b: (b, 0, 0)),
            pl.BlockSpec((_GD, _C), lambda b: (0, 0)),
            pl.BlockSpec((1, _GD), lambda b: (0, 0)),
            pl.BlockSpec((_C, _GD), lambda b: (0, 0)),
            pl.BlockSpec((1, _C), lambda b: (0, 0)),
        ],
        out_specs=pl.BlockSpec((_NB, _NT, _C), lambda b: (b, 0, 0)),
        out_shape=jax.ShapeDtypeStruct((B, _NT, _C), jnp.float32),
        scratch_shapes=[pltpu.VMEM((_NB * _NT, _C), jnp.float32),
                        pltpu.SemaphoreType.DMA],
    )(z, x, w_down, bd, w_up, bu)


# speedup vs baseline: 2.2312x; 2.2312x over previous
import jax
import jax.numpy as jnp
from jax import lax
from jax.experimental import pallas as pl
from jax.experimental.pallas import tpu as pltpu

_C = 384
_NS = 1024
_WS = 4
_WNH = 8
_NW = 64
_K = 16
_G = 24
_GD = 4 * _G
_NT = _K * _WS * _WS
_NB = 8

_RHS_T = (((1,), (1,)), ((), ()))


def _body(z_ref, x_ref, wd_ref, bd_ref, wu_ref, bu_ref, out_ref, xe_ref):
    ti = lax.broadcasted_iota(jnp.int32, (_NW, _NS), 1)
    wi = lax.broadcasted_iota(jnp.int32, (_NW, _NS), 0)
    q_sel = ((ti // 128 == wi // _WNH)
             & ((ti % 32) // _WS == wi % _WNH)).astype(jnp.float32)
    eye = (lax.broadcasted_iota(jnp.int32, (_NW, _NW), 0)
           == lax.broadcasted_iota(jnp.int32, (_NW, _NW), 1)).astype(
        jnp.float32)
    wi64 = lax.broadcasted_iota(jnp.int32, (_NW, _NW), 0)
    ji64 = lax.broadcasted_iota(jnp.int32, (_NW, _NW), 1)
    fidx = lax.broadcasted_iota(jnp.int32, (_NW, 1), 0)

    zmax8 = jnp.concatenate(
        [jnp.max(z_ref[i], axis=0, keepdims=True) for i in range(_NB)],
        axis=0)
    xall = x_ref[...].reshape(_NB * _NS, _C)
    sim_all = lax.dot_general(zmax8, xall, _RHS_T,
                              preferred_element_type=jnp.float32)

    for i in range(_NB):
        sim = lax.slice(sim_all, (i, _NS * i), (i + 1, _NS * (i + 1)))
        win_row = lax.dot_general(sim, q_sel, _RHS_T,
                                  preferred_element_type=jnp.float32,
                                  precision=lax.Precision.HIGHEST)
        win_col = lax.dot_general(eye, win_row, _RHS_T,
                                  preferred_element_type=jnp.float32,
                                  precision=lax.Precision.HIGHEST)
        vj = jnp.broadcast_to(win_row, (_NW, _NW))
        vw = jnp.broadcast_to(win_col, (_NW, _NW))
        beats = (vj > vw) | ((vj == vw) & (ji64 < wi64))
        rank = jnp.sum(beats.astype(jnp.int32), axis=1, keepdims=True)
        for k in range(_K):
            w = jnp.sum(jnp.where(rank == k, fidx, 0))
            wr = w // _WNH
            base = wr * 128 + (w - wr * _WNH) * _WS
            row0 = i * _NT + _K * k
            for r in range(_WS):
                for c in range(_WS):
                    xe_ref[pl.ds(row0 + _WS * r + c, 1), :] = (
                        x_ref[i, pl.ds(base + 32 * r + c, 1), :])

    rows = _NB * _NT
    xe = xe_ref[...]
    wd = wd_ref[...]
    bd = bd_ref[...]
    wu = wu_ref[...]
    bu = bu_ref[...]

    t = lax.dot_general(xe, wd, _RHS_T,
                        preferred_element_type=jnp.float32) + bd

    qi = lax.broadcasted_iota(jnp.int32, (rows, _GD), 0)
    li = lax.broadcasted_iota(jnp.int32, (rows, _GD), 1)
    c_tok = qi % _WS
    r_tok = (qi // _WS) % _WS
    z1 = jnp.zeros((1, _GD), jnp.float32)
    z4 = jnp.zeros((_WS, _GD), jnp.float32)
    tm1 = jnp.concatenate([t[1:], z1], axis=0)
    tp1 = jnp.concatenate([z1, t[:-1]], axis=0)
    tm4 = jnp.concatenate([t[_WS:], z4], axis=0)
    tp4 = jnp.concatenate([z4, t[:-_WS]], axis=0)
    g0 = jnp.where(c_tok < _WS - 1, tm1, 0.0)
    g1 = jnp.where(c_tok > 0, tp1, 0.0)
    g2 = jnp.where(r_tok < _WS - 1, tm4, 0.0)
    g3 = jnp.where(r_tok > 0, tp4, 0.0)
    s = jnp.where(li < _G, g0,
                  jnp.where(li < 2 * _G, g1,
                            jnp.where(li < 3 * _G, g2, g3)))
    su = lax.dot_general(s, wu, _RHS_T,
                         preferred_element_type=jnp.float32)
    out = xe + su + bu
    for i in range(_NB):
        out_ref[i] = out[i * _NT:(i + 1) * _NT]


def kernel(z, x, w_down, b_down, w_up, b_up):
    B = z.shape[0]
    bd = b_down.reshape(1, _GD)
    bu = b_up.reshape(1, _C)
    return pl.pallas_call(
        _body,
        grid=(B // _NB,),
        in_specs=[
            pl.BlockSpec((_NB, z.shape[1], _C), lambda b: (b, 0, 0)),
            pl.BlockSpec((_NB, _NS, _C), lambda b: (b, 0, 0)),
            pl.BlockSpec((_GD, _C), lambda b: (0, 0)),
            pl.BlockSpec((1, _GD), lambda b: (0, 0)),
            pl.BlockSpec((_C, _GD), lambda b: (0, 0)),
            pl.BlockSpec((1, _C), lambda b: (0, 0)),
        ],
        out_specs=pl.BlockSpec((_NB, _NT, _C), lambda b: (b, 0, 0)),
        out_shape=jax.ShapeDtypeStruct((B, _NT, _C), jnp.float32),
        scratch_shapes=[pltpu.VMEM((_NB * _NT, _C), jnp.float32)],
    )(z, x, w_down, bd, w_up, bu)


# speedup vs baseline: 2.4423x; 1.0946x over previous
import jax
import jax.numpy as jnp
from jax import lax
from jax.experimental import pallas as pl
from jax.experimental.pallas import tpu as pltpu

_C = 384
_NS = 1024
_WS = 4
_WNH = 8
_NW = 64
_K = 16
_G = 24
_GD = 4 * _G
_NT = _K * _WS * _WS
_NB = 8

_RHS_T = (((1,), (1,)), ((), ()))


def _body(z_ref, x_ref, wd_ref, bd_ref, wu_ref, bu_ref, out_ref, xe_ref):
    ti = lax.broadcasted_iota(jnp.int32, (_NW, _NS), 1)
    wi = lax.broadcasted_iota(jnp.int32, (_NW, _NS), 0)
    q_sel = ((ti // 128 == wi // _WNH)
             & ((ti % 32) // _WS == wi % _WNH)).astype(jnp.float32)
    eye = (lax.broadcasted_iota(jnp.int32, (_NW, _NW), 0)
           == lax.broadcasted_iota(jnp.int32, (_NW, _NW), 1)).astype(
        jnp.float32)
    wi64 = lax.broadcasted_iota(jnp.int32, (_NW, _NW), 0)
    ji64 = lax.broadcasted_iota(jnp.int32, (_NW, _NW), 1)
    fidx = lax.broadcasted_iota(jnp.int32, (_NW, 1), 0)

    for i in range(_NB):
        z = z_ref[i]
        zmax = jnp.max(z, axis=0, keepdims=True)
        sim = lax.dot_general(zmax, x_ref[i], _RHS_T,
                              preferred_element_type=jnp.float32)
        win_row = lax.dot_general(sim, q_sel, _RHS_T,
                                  preferred_element_type=jnp.float32,
                                  precision=lax.Precision.HIGHEST)
        win_col = lax.dot_general(eye, win_row, _RHS_T,
                                  preferred_element_type=jnp.float32,
                                  precision=lax.Precision.HIGHEST)
        vj = jnp.broadcast_to(win_row, (_NW, _NW))
        vw = jnp.broadcast_to(win_col, (_NW, _NW))
        beats = (vj > vw) | ((vj == vw) & (ji64 < wi64))
        rank = jnp.sum(beats.astype(jnp.int32), axis=1, keepdims=True)
        for k in range(_K):
            w = jnp.sum(jnp.where(rank == k, fidx, 0))
            wr = w // _WNH
            base = wr * 128 + (w - wr * _WNH) * _WS
            row0 = i * _NT + _K * k
            for r in range(_WS):
                for c in range(_WS):
                    xe_ref[pl.ds(row0 + _WS * r + c, 1), :] = (
                        x_ref[i, pl.ds(base + 32 * r + c, 1), :])

    rows = _NB * _NT
    xe = xe_ref[...]
    wd = wd_ref[...]
    bd = bd_ref[...]
    wu = wu_ref[...]
    bu = bu_ref[...]

    t = lax.dot_general(xe, wd, _RHS_T,
                        preferred_element_type=jnp.float32) + bd

    qi = lax.broadcasted_iota(jnp.int32, (rows, _GD), 0)
    li = lax.broadcasted_iota(jnp.int32, (rows, _GD), 1)
    c_tok = qi % _WS
    r_tok = (qi // _WS) % _WS
    z1 = jnp.zeros((1, _GD), jnp.float32)
    z4 = jnp.zeros((_WS, _GD), jnp.float32)
    tm1 = jnp.concatenate([t[1:], z1], axis=0)
    tp1 = jnp.concatenate([z1, t[:-1]], axis=0)
    tm4 = jnp.concatenate([t[_WS:], z4], axis=0)
    tp4 = jnp.concatenate([z4, t[:-_WS]], axis=0)
    g0 = jnp.where(c_tok < _WS - 1, tm1, 0.0)
    g1 = jnp.where(c_tok > 0, tp1, 0.0)
    g2 = jnp.where(r_tok < _WS - 1, tm4, 0.0)
    g3 = jnp.where(r_tok > 0, tp4, 0.0)
    s = jnp.where(li < _G, g0,
                  jnp.where(li < 2 * _G, g1,
                            jnp.where(li < 3 * _G, g2, g3)))
    su = lax.dot_general(s, wu, _RHS_T,
                         preferred_element_type=jnp.float32)
    out = xe + su + bu
    for i in range(_NB):
        out_ref[i] = out[i * _NT:(i + 1) * _NT]


def kernel(z, x, w_down, b_down, w_up, b_up):
    B = z.shape[0]
    bd = b_down.reshape(1, _GD)
    bu = b_up.reshape(1, _C)
    return pl.pallas_call(
        _body,
        grid=(B // _NB,),
        in_specs=[
            pl.BlockSpec((_NB, z.shape[1], _C), lambda b: (b, 0, 0)),
            pl.BlockSpec((_NB, _NS, _C), lambda b: (b, 0, 0)),
            pl.BlockSpec((_GD, _C), lambda b: (0, 0)),
            pl.BlockSpec((1, _GD), lambda b: (0, 0)),
            pl.BlockSpec((_C, _GD), lambda b: (0, 0)),
            pl.BlockSpec((1, _C), lambda b: (0, 0)),
        ],
        out_specs=pl.BlockSpec((_NB, _NT, _C), lambda b: (b, 0, 0)),
        out_shape=jax.ShapeDtypeStruct((B, _NT, _C), jnp.float32),
        scratch_shapes=[pltpu.VMEM((_NB * _NT, _C), jnp.float32)],
    )(z, x, w_down, bd, w_up, bu)
